# baseline (speedup 1.0000x reference)
import functools

import jax
import jax.numpy as jnp
from jax import lax
from jax.experimental import pallas as pl
from jax.experimental.pallas import tpu as pltpu
from jax.experimental.pallas import tpu_sc as plsc

N = 1024
E = 32768
H = 128
C = 16
L = 3
EN = E + N
P = N * (N - 1) // 2


def _mm(a, b):
    return jnp.dot(a, b, precision=jax.lax.Precision.HIGHEST,
                   preferred_element_type=jnp.float32)

_NUM_W = 32
EPW = 1152
NCH = EPW // 128
E_PAD = _NUM_W * EPW
RPS = N // 16


def _agg_sc_body(table_hbm, src_hbm, dst_hbm, zeros_hbm, out_hbm,
                 s0, s1, s2, s3, s4, s5, s6, s7, s8,
                 didx_v, rows0, rows1, sem0, sem1, shared, shtab):
    sidx = (s0, s1, s2, s3, s4, s5, s6, s7, s8)
    info = plsc.get_sparse_core_info()
    nc = info.num_cores
    cid = lax.axis_index("c")
    sid = lax.axis_index("s")
    wid = sid * nc + cid

    pltpu.sync_copy(table_hbm.at[pl.ds(sid * RPS, RPS)],
                    shtab.at[pl.ds(sid * RPS, RPS)])
    pltpu.sync_copy(zeros_hbm, shared.at[pl.ds(sid * RPS, RPS)])
    for ch in range(NCH):
        pltpu.sync_copy(
            src_hbm.at[pl.ds((wid * NCH + ch) * 128, 128)], sidx[ch])
    pltpu.sync_copy(dst_hbm.at[wid], didx_v)
    plsc.subcore_barrier()

    bufs = (rows0, rows1)
    sems = (sem0, sem1)
    cps = [None] * NCH
    cps[0] = pltpu.async_copy(shtab.at[sidx[0]], bufs[0], sems[0])
    for ch in range(NCH):
        if ch + 1 < NCH:
            cps[ch + 1] = pltpu.async_copy(
                shtab.at[sidx[ch + 1]], bufs[(ch + 1) % 2],
                sems[(ch + 1) % 2])
        cps[ch].wait()
        pltpu.sync_copy(bufs[ch % 2], shared.at[didx_v.at[ch]], add=True)
    plsc.subcore_barrier()

    pltpu.sync_copy(shared.at[pl.ds(sid * RPS, RPS)],
                    out_hbm.at[cid, pl.ds(sid * RPS, RPS)])


@functools.cache
def _agg_kernel(width):
    return functools.partial(
        pl.kernel,
        mesh=plsc.VectorSubcoreMesh(core_axis_name="c", subcore_axis_name="s"),
        out_type=jax.ShapeDtypeStruct((2, N, width), jnp.float32),
        scratch_types=[
            *([pltpu.VMEM((128,), jnp.int32)] * NCH),
            pltpu.VMEM((NCH, 128), jnp.int32),
            pltpu.VMEM((128, width), jnp.float32),
            pltpu.VMEM((128, width), jnp.float32),
            pltpu.SemaphoreType.DMA,
            pltpu.SemaphoreType.DMA,
            pltpu.VMEM_SHARED((N + 8, width), jnp.float32),
            pltpu.VMEM_SHARED((N, width), jnp.float32),
        ],
    )(_agg_sc_body)


DBLK = 512


def _pre_body(Y_ref, t_ref, emb_ref, Wt1_ref, Wt2_ref, Win_ref, Wm1_ref,
              Wm2_ref, dst_ref, h_ref, m_ref, rdeg_ref):
    onehot = (Y_ref[:] == lax.broadcasted_iota(jnp.int32, (N, C), 1))
    h0 = _mm(onehot.astype(jnp.float32), emb_ref[:])
    t_emb = _mm(jnp.maximum(t_ref[:] * Wt1_ref[:], 0.0), Wt2_ref[:])
    h = jnp.maximum(_mm(h0 + t_emb, Win_ref[:]), 0.0)
    h_ref[:] = h
    m_ref[:] = _mm(jnp.maximum(_mm(h, Wm1_ref[0]), 0.0), Wm2_ref[0])
    lanes = lax.broadcasted_iota(jnp.int32, (DBLK, N), 1)
    deg_row = jnp.zeros((1, N), jnp.float32)
    for g in range(E_PAD // DBLK):
        d = dst_ref[pl.ds(g * DBLK, DBLK), :]
        deg_row = deg_row + jnp.sum(
            (d == lanes).astype(jnp.float32), axis=0, keepdims=True)
    rdeg_ref[:] = 1.0 / jnp.maximum(deg_row, 1.0).reshape(N, 1)


def _layer_update(h_ref, g0_ref, g1_ref, rdeg_ref, Wu_ref, i):
    agg = (g0_ref[:] + g1_ref[:]) * rdeg_ref[:]
    h = h_ref[:]
    upd = jnp.maximum(_mm(h, Wu_ref[i, :H, :]) + _mm(agg, Wu_ref[i, H:, :]), 0.0)
    return h + upd


def _make_layer_body(i):
    def body(h_ref, g0_ref, g1_ref, rdeg_ref, Wu_ref, Wm1_ref,
             Wm2_ref, h2_ref, m2_ref):
        h2 = _layer_update(h_ref, g0_ref, g1_ref, rdeg_ref, Wu_ref, i)
        h2_ref[:] = h2
        m2_ref[:] = _mm(jnp.maximum(_mm(h2, Wm1_ref[i + 1]), 0.0), Wm2_ref[i + 1])

    return body


def _final_body(h_ref, g0_ref, g1_ref, rdeg_ref, Wu_ref, We1_ref,
                Ae_ref, Be_ref):
    h2 = _layer_update(h_ref, g0_ref, g1_ref, rdeg_ref, Wu_ref, L - 1)
    Ae_ref[:] = _mm(h2, We1_ref[:H, :])
    Be_ref[:] = _mm(h2, We1_ref[H:, :])


BI = 64
BJ = 128


def _pairs_body(Ae_ref, Be_ref, We2_ref, We3_ref, L0_ref, L1_ref):
    a = Ae_ref[:]
    b = Be_ref[:]
    z = jnp.maximum(a[:, None, :] + b[None, :, :], 0.0)
    z2 = z.reshape(BI * BJ, H)
    o = jnp.maximum(_mm(z2, We2_ref[:]), 0.0)
    lg = _mm(o, We3_ref[:])
    lg3 = lg.reshape(BI, BJ, 2)
    L0_ref[:] = lg3[:, :, 0]
    L1_ref[:] = lg3[:, :, 1]


LROWS = (N + 8) * N // 128
OROWS = 4104
P_PAD = OROWS * 128


def _seg_params():
    params = []
    for i in range(N - 1):
        d = i * (N - 1) - (i * (i - 1)) // 2
        s = (N + 1) * i + 1
        params.append((s >> 7, s & 127, d >> 7, d & 127))
    return params


_SEGS = _seg_params()


def _extract_body(L0_ref, L1_ref, O0_ref, O1_ref):
    lane = lax.broadcasted_iota(jnp.int32, (8, 128), 1)
    lane9 = lax.broadcasted_iota(jnp.int32, (9, 128), 1)
    for (B, t, dr, dq) in _SEGS:
        for (L_ref, O_ref) in ((L0_ref, O0_ref), (L1_ref, O1_ref)):
            srcw = L_ref[pl.ds(B, 9), :]
            a = pltpu.roll(srcw[:8, :], (128 - t) % 128, axis=1)
            b = pltpu.roll(srcw[1:, :], (128 - t) % 128, axis=1)
            w8 = jnp.where(lane < 128 - t, a, b)
            wa = pltpu.roll(w8, dq, axis=1)
            cur = O_ref[pl.ds(dr, 1), :]
            hi = jnp.concatenate([wa, wa[7:8, :]], axis=0)
            lo = jnp.concatenate([cur, wa], axis=0)
            O_ref[pl.ds(dr, 9), :] = jnp.where(lane9 >= dq, hi, lo)


def kernel(edge_index, Y, t_normalized, emb, Wt1, Wt2, Win, Wm1, Wm2, Wu,
           We1, We2, We3):
    self_idx = jnp.arange(N, dtype=edge_index.dtype)
    pad = E_PAD - EN
    src_all = jnp.concatenate(
        [edge_index[0], self_idx,
         jnp.zeros((pad,), edge_index.dtype)])
    dst_all = jnp.concatenate(
        [edge_index[1], self_idx,
         jnp.full((pad,), N, edge_index.dtype)]).reshape(_NUM_W, NCH, 128)
    zeros_n = jnp.zeros((RPS, H), jnp.float32)

    agg_n = _agg_kernel(H)
    nh = jax.ShapeDtypeStruct((N, H), jnp.float32)

    h, m, rdeg = pl.pallas_call(
        _pre_body,
        out_shape=[nh, nh, jax.ShapeDtypeStruct((N, 1), jnp.float32)],
    )(Y.reshape(N, 1), t_normalized.reshape(1, 1), emb, Wt1, Wt2, Win,
      Wm1, Wm2, dst_all.reshape(E_PAD, 1))

    for i in range(L - 1):
        G = agg_n(m, src_all, dst_all, zeros_n)
        h, m = pl.pallas_call(
            _make_layer_body(i), out_shape=[nh, nh],
        )(h, G[0], G[1], rdeg, Wu, Wm1, Wm2)
    G = agg_n(m, src_all, dst_all, zeros_n)
    Ae, Be = pl.pallas_call(
        _final_body, out_shape=[nh, nh],
    )(h, G[0], G[1], rdeg, Wu, We1)

    L0, L1 = pl.pallas_call(
        _pairs_body,
        grid=(N // BI, N // BJ),
        in_specs=[
            pl.BlockSpec((BI, H), lambda ib, jb: (ib, 0)),
            pl.BlockSpec((BJ, H), lambda ib, jb: (jb, 0)),
            pl.BlockSpec((H, H), lambda ib, jb: (0, 0)),
            pl.BlockSpec((H, 2), lambda ib, jb: (0, 0)),
        ],
        out_specs=[
            pl.BlockSpec((BI, BJ), lambda ib, jb: (ib, jb)),
            pl.BlockSpec((BI, BJ), lambda ib, jb: (ib, jb)),
        ],
        out_shape=[
            jax.ShapeDtypeStruct((N + 8, N), jnp.float32),
            jax.ShapeDtypeStruct((N + 8, N), jnp.float32),
        ],
    )(Ae, Be, We2, We3)

    O0, O1 = pl.pallas_call(
        _extract_body,
        out_shape=[
            jax.ShapeDtypeStruct((OROWS, 128), jnp.float32),
            jax.ShapeDtypeStruct((OROWS, 128), jnp.float32),
        ],
    )(L0.reshape(LROWS, 128), L1.reshape(LROWS, 128))

    return jnp.stack(
        [O0.reshape(P_PAD)[:P], O1.reshape(P_PAD)[:P]], axis=1)


# speedup vs baseline: 3.1262x; 3.1262x over previous
import functools

import jax
import jax.numpy as jnp
from jax import lax
from jax.experimental import pallas as pl
from jax.experimental.pallas import tpu as pltpu
from jax.experimental.pallas import tpu_sc as plsc

N = 1024
E = 32768
H = 128
C = 16
L = 3
EN = E + N
P = N * (N - 1) // 2


def _mm(a, b):
    return jnp.dot(a, b, precision=jax.lax.Precision.HIGHEST,
                   preferred_element_type=jnp.float32)

_NUM_W = 32
EPW = 1152
NCH = EPW // 128
E_PAD = _NUM_W * EPW
RPS = N // 16


def _agg_sc_body(table_hbm, src_hbm, dst_hbm, zeros_hbm, out_hbm,
                 s0, s1, s2, s3, s4, s5, s6, s7, s8,
                 didx_v, rows0, rows1, sem0, sem1, shared, shtab):
    sidx = (s0, s1, s2, s3, s4, s5, s6, s7, s8)
    info = plsc.get_sparse_core_info()
    nc = info.num_cores
    cid = lax.axis_index("c")
    sid = lax.axis_index("s")
    wid = sid * nc + cid

    pltpu.sync_copy(table_hbm.at[pl.ds(sid * RPS, RPS)],
                    shtab.at[pl.ds(sid * RPS, RPS)])
    pltpu.sync_copy(zeros_hbm, shared.at[pl.ds(sid * RPS, RPS)])
    for ch in range(NCH):
        pltpu.sync_copy(
            src_hbm.at[pl.ds((wid * NCH + ch) * 128, 128)], sidx[ch])
    pltpu.sync_copy(dst_hbm.at[wid], didx_v)
    plsc.subcore_barrier()

    bufs = (rows0, rows1)
    sems = (sem0, sem1)
    cps = [None] * NCH
    cps[0] = pltpu.async_copy(shtab.at[sidx[0]], bufs[0], sems[0])
    for ch in range(NCH):
        if ch + 1 < NCH:
            cps[ch + 1] = pltpu.async_copy(
                shtab.at[sidx[ch + 1]], bufs[(ch + 1) % 2],
                sems[(ch + 1) % 2])
        cps[ch].wait()
        pltpu.sync_copy(bufs[ch % 2], shared.at[didx_v.at[ch]], add=True)
    plsc.subcore_barrier()

    pltpu.sync_copy(shared.at[pl.ds(sid * RPS, RPS)],
                    out_hbm.at[cid, pl.ds(sid * RPS, RPS)])


@functools.cache
def _agg_kernel(width):
    return functools.partial(
        pl.kernel,
        mesh=plsc.VectorSubcoreMesh(core_axis_name="c", subcore_axis_name="s"),
        out_type=jax.ShapeDtypeStruct((2, N, width), jnp.float32),
        scratch_types=[
            *([pltpu.VMEM((128,), jnp.int32)] * NCH),
            pltpu.VMEM((NCH, 128), jnp.int32),
            pltpu.VMEM((128, width), jnp.float32),
            pltpu.VMEM((128, width), jnp.float32),
            pltpu.SemaphoreType.DMA,
            pltpu.SemaphoreType.DMA,
            pltpu.VMEM_SHARED((N + 8, width), jnp.float32),
            pltpu.VMEM_SHARED((N, width), jnp.float32),
        ],
    )(_agg_sc_body)


DBLK = 512


def _pre_body(Y_ref, t_ref, emb_ref, Wt1_ref, Wt2_ref, Win_ref, Wm1_ref,
              Wm2_ref, dst_ref, h_ref, m_ref, rdeg_ref):
    onehot = (Y_ref[:] == lax.broadcasted_iota(jnp.int32, (N, C), 1))
    h0 = _mm(onehot.astype(jnp.float32), emb_ref[:])
    t_emb = _mm(jnp.maximum(t_ref[:] * Wt1_ref[:], 0.0), Wt2_ref[:])
    h = jnp.maximum(_mm(h0 + t_emb, Win_ref[:]), 0.0)
    h_ref[:] = h
    m_ref[:] = _mm(jnp.maximum(_mm(h, Wm1_ref[0]), 0.0), Wm2_ref[0])
    lanes = lax.broadcasted_iota(jnp.int32, (DBLK, N), 1)
    deg_row = jnp.zeros((1, N), jnp.float32)
    for g in range(E_PAD // DBLK):
        d = dst_ref[pl.ds(g * DBLK, DBLK), :]
        deg_row = deg_row + jnp.sum(
            (d == lanes).astype(jnp.float32), axis=0, keepdims=True)
    rdeg_ref[:] = 1.0 / jnp.maximum(deg_row, 1.0).reshape(N, 1)


def _layer_update(h_ref, g0_ref, g1_ref, rdeg_ref, Wu_ref, i):
    agg = (g0_ref[:] + g1_ref[:]) * rdeg_ref[:]
    h = h_ref[:]
    upd = jnp.maximum(_mm(h, Wu_ref[i, :H, :]) + _mm(agg, Wu_ref[i, H:, :]), 0.0)
    return h + upd


def _make_layer_body(i):
    def body(h_ref, g0_ref, g1_ref, rdeg_ref, Wu_ref, Wm1_ref,
             Wm2_ref, h2_ref, m2_ref):
        h2 = _layer_update(h_ref, g0_ref, g1_ref, rdeg_ref, Wu_ref, i)
        h2_ref[:] = h2
        m2_ref[:] = _mm(jnp.maximum(_mm(h2, Wm1_ref[i + 1]), 0.0), Wm2_ref[i + 1])

    return body


def _final_body(h_ref, g0_ref, g1_ref, rdeg_ref, Wu_ref, We1_ref,
                Ae_ref, Be_ref):
    h2 = _layer_update(h_ref, g0_ref, g1_ref, rdeg_ref, Wu_ref, L - 1)
    Ae_ref[:] = _mm(h2, We1_ref[:H, :])
    Be_ref[:] = _mm(h2, We1_ref[H:, :])


BI = 64
BJ = 128
_UT_BLOCKS = [(ib, jb) for ib in range(N // BI) for jb in range(N // BJ)
              if (jb + 1) * BJ - 1 > ib * BI]
import numpy as _np
_UT_IBS = _np.array([b[0] for b in _UT_BLOCKS], _np.int32)
_UT_JBS = _np.array([b[1] for b in _UT_BLOCKS], _np.int32)


def _pairs_body(ib_ref, jb_ref, Ae_ref, Be_ref, We2_ref, We3_ref,
                L0_ref, L1_ref):
    del ib_ref, jb_ref
    a = Ae_ref[:]
    b = Be_ref[:]
    z = jnp.maximum(a[:, None, :] + b[None, :, :], 0.0)
    z2 = z.reshape(BI * BJ, H)
    o = jnp.maximum(_mm(z2, We2_ref[:]), 0.0)
    w3 = We3_ref[:]
    lg0 = jnp.sum(o * w3[:, 0][None, :], axis=1)
    lg1 = jnp.sum(o * w3[:, 1][None, :], axis=1)
    L0_ref[:] = lg0.reshape(BI, BJ)
    L1_ref[:] = lg1.reshape(BI, BJ)


LROWS = (N + 8) * N // 128
OROWS = 4104
P_PAD = OROWS * 128


def _seg_params():
    params = []
    for i in range(N - 1):
        d = i * (N - 1) - (i * (i - 1)) // 2
        s = (N + 1) * i + 1
        params.append((s >> 7, s & 127, d >> 7, d & 127))
    return params


_SEGS = _seg_params()


def _extract_body(L0_ref, L1_ref, O0_ref, O1_ref):
    lane = lax.broadcasted_iota(jnp.int32, (8, 128), 1)
    lane9 = lax.broadcasted_iota(jnp.int32, (9, 128), 1)
    for (B, t, dr, dq) in _SEGS:
        for (L_ref, O_ref) in ((L0_ref, O0_ref), (L1_ref, O1_ref)):
            srcw = L_ref[pl.ds(B, 9), :]
            a = pltpu.roll(srcw[:8, :], (128 - t) % 128, axis=1)
            b = pltpu.roll(srcw[1:, :], (128 - t) % 128, axis=1)
            w8 = jnp.where(lane < 128 - t, a, b)
            wa = pltpu.roll(w8, dq, axis=1)
            cur = O_ref[pl.ds(dr, 1), :]
            hi = jnp.concatenate([wa, wa[7:8, :]], axis=0)
            lo = jnp.concatenate([cur, wa], axis=0)
            O_ref[pl.ds(dr, 9), :] = jnp.where(lane9 >= dq, hi, lo)


def kernel(edge_index, Y, t_normalized, emb, Wt1, Wt2, Win, Wm1, Wm2, Wu,
           We1, We2, We3):
    self_idx = jnp.arange(N, dtype=edge_index.dtype)
    pad = E_PAD - EN
    src_all = jnp.concatenate(
        [edge_index[0], self_idx,
         jnp.zeros((pad,), edge_index.dtype)])
    dst_all = jnp.concatenate(
        [edge_index[1], self_idx,
         jnp.full((pad,), N, edge_index.dtype)]).reshape(_NUM_W, NCH, 128)
    zeros_n = jnp.zeros((RPS, H), jnp.float32)

    agg_n = _agg_kernel(H)
    nh = jax.ShapeDtypeStruct((N, H), jnp.float32)

    h, m, rdeg = pl.pallas_call(
        _pre_body,
        out_shape=[nh, nh, jax.ShapeDtypeStruct((N, 1), jnp.float32)],
    )(Y.reshape(N, 1), t_normalized.reshape(1, 1), emb, Wt1, Wt2, Win,
      Wm1, Wm2, dst_all.reshape(E_PAD, 1))

    for i in range(L - 1):
        G = agg_n(m, src_all, dst_all, zeros_n)
        h, m = pl.pallas_call(
            _make_layer_body(i), out_shape=[nh, nh],
        )(h, G[0], G[1], rdeg, Wu, Wm1, Wm2)
    G = agg_n(m, src_all, dst_all, zeros_n)
    Ae, Be = pl.pallas_call(
        _final_body, out_shape=[nh, nh],
    )(h, G[0], G[1], rdeg, Wu, We1)

    L0, L1 = pl.pallas_call(
        _pairs_body,
        grid_spec=pltpu.PrefetchScalarGridSpec(
            num_scalar_prefetch=2,
            grid=(len(_UT_BLOCKS),),
            in_specs=[
                pl.BlockSpec((BI, H), lambda g, ibs, jbs: (ibs[g], 0)),
                pl.BlockSpec((BJ, H), lambda g, ibs, jbs: (jbs[g], 0)),
                pl.BlockSpec((H, H), lambda g, ibs, jbs: (0, 0)),
                pl.BlockSpec((H, 2), lambda g, ibs, jbs: (0, 0)),
            ],
            out_specs=[
                pl.BlockSpec((BI, BJ), lambda g, ibs, jbs: (ibs[g], jbs[g])),
                pl.BlockSpec((BI, BJ), lambda g, ibs, jbs: (ibs[g], jbs[g])),
            ],
        ),
        out_shape=[
            jax.ShapeDtypeStruct((N + 8, N), jnp.float32),
            jax.ShapeDtypeStruct((N + 8, N), jnp.float32),
        ],
    )(_UT_IBS, _UT_JBS, Ae, Be, We2, We3)

    O0, O1 = pl.pallas_call(
        _extract_body,
        out_shape=[
            jax.ShapeDtypeStruct((OROWS, 128), jnp.float32),
            jax.ShapeDtypeStruct((OROWS, 128), jnp.float32),
        ],
    )(L0.reshape(LROWS, 128), L1.reshape(LROWS, 128))

    return jnp.stack(
        [O0.reshape(P_PAD)[:P], O1.reshape(P_PAD)[:P]], axis=1)
